# baseline (speedup 1.0000x reference)
import functools

import jax
import jax.numpy as jnp
from jax import lax
from jax.experimental import pallas as pl
from jax.experimental.pallas import tpu as pltpu
from jax.experimental.pallas import tpu_sc as plsc

_ERRP = 0.05
_V = 64
_NC = 2
_NS = 16
_LANES = 16
_CHUNK_ROWS = 200


def _packed_dest_table(bl):
    key = jax.random.key(42)
    k1, k2 = jax.random.split(key)
    tm = jax.random.uniform(k1, (bl, _V - 1)) < _ERRP
    repl_ids = jax.random.randint(k2, (bl, _V - 1), 0, _V - 2)
    s = jnp.arange(1, _V, dtype=jnp.int32)[None, :]
    repl_sym = jnp.where(repl_ids + 1 < s, repl_ids + 1, repl_ids + 2)
    dest = jnp.where(tm, repl_sym, s).astype(jnp.int32)
    dest = jnp.concatenate([jnp.zeros((bl, 1), jnp.int32), dest], axis=1)
    d4 = dest.reshape(bl, 4, _LANES)
    shifts = jnp.array([0, 8, 16, 24], jnp.int32)[None, :, None]
    return jnp.sum(d4 << shifts, axis=1, dtype=jnp.int32)


def _sc_scatter_body(
    m_hbm, pk_hbm, out_hbm,
    val0, val1, pk0, pk1, acc0, acc1,
    vs0, vs1, ps0, ps1, os0, os1,
):
    nw = _NC * _NS
    wid = lax.axis_index("s") * _NC + lax.axis_index("c")
    total = m_hbm.shape[0]
    per_w = total // nw
    chunk = _CHUNK_ROWS * _V
    pk_chunk = _CHUNK_ROWS * _LANES
    n_chunks = per_w // chunk
    wrow0 = wid * (per_w // _V)

    val = (val0, val1)
    pkb = (pk0, pk1)
    acc = (acc0, acc1)
    vsem = (vs0, vs1)
    psem = (ps0, ps1)
    osem = (os0, os1)

    def in_val(c, b):
        row0 = wrow0 + c * _CHUNK_ROWS
        return pltpu.make_async_copy(
            m_hbm.at[pl.ds(row0 * _V, chunk)], val[b], vsem[b]
        )

    def in_pk(c, b):
        row0 = wrow0 + c * _CHUNK_ROWS
        return pltpu.make_async_copy(
            pk_hbm.at[pl.ds(row0 * _LANES, pk_chunk)], pkb[b], psem[b]
        )

    def out_acc(c, b):
        row0 = wrow0 + c * _CHUNK_ROWS
        return pltpu.make_async_copy(
            acc[b], out_hbm.at[pl.ds(row0 * _V, chunk)], osem[b]
        )

    for b in range(2):
        in_val(b, b).start()
        in_pk(b, b).start()

    def compute(b):
        def row_body(r, carry):
            rb = r * _V
            for q in range(_V // _LANES):
                acc[b][pl.ds(rb + q * _LANES, _LANES)] = jnp.zeros(
                    (_LANES,), jnp.float32
                )
            pk = pkb[b][pl.ds(r * _LANES, _LANES)]
            for q in range(_V // _LANES):
                idx = ((pk >> (8 * q)) & 63) + rb
                v = val[b][pl.ds(rb + q * _LANES, _LANES)]
                plsc.addupdate_scatter(acc[b], [idx], v)
            return carry

        lax.fori_loop(0, _CHUNK_ROWS, row_body, 0, unroll=4)

    def pair_body(g, carry):
        for b in range(2):
            c = 2 * g + b
            in_val(c, b).wait()
            in_pk(c, b).wait()

            @pl.when(c >= 2)
            def _drain():
                out_acc(c - 2, b).wait()

            compute(b)

            @pl.when(c + 2 < n_chunks)
            def _prefetch():
                in_val(c + 2, b).start()
                in_pk(c + 2, b).start()

            out_acc(c, b).start()
        return carry

    lax.fori_loop(0, n_chunks // 2, pair_body, 0)
    for b in range(2):
        out_acc(n_chunks - 2 + b, b).wait()


def _sc_scatter(m_flat, packed_dest):
    total = m_flat.shape[0]
    chunk = _CHUNK_ROWS * _V
    mesh = plsc.VectorSubcoreMesh(core_axis_name="c", subcore_axis_name="s")
    return pl.kernel(
        _sc_scatter_body,
        mesh=mesh,
        out_type=jax.ShapeDtypeStruct((total,), jnp.float32),
        scratch_types=[
            pltpu.VMEM((chunk,), jnp.float32),
            pltpu.VMEM((chunk,), jnp.float32),
            pltpu.VMEM((_CHUNK_ROWS * _LANES,), jnp.int32),
            pltpu.VMEM((_CHUNK_ROWS * _LANES,), jnp.int32),
            pltpu.VMEM((chunk,), jnp.float32),
            pltpu.VMEM((chunk,), jnp.float32),
            pltpu.SemaphoreType.DMA,
            pltpu.SemaphoreType.DMA,
            pltpu.SemaphoreType.DMA,
            pltpu.SemaphoreType.DMA,
            pltpu.SemaphoreType.DMA,
            pltpu.SemaphoreType.DMA,
        ],
        compiler_params=pltpu.CompilerParams(needs_layout_passes=False),
    )(m_flat, packed_dest)


def _tc_rest_body(p_ref, m_ref, po_ref, pc_ref, mc_ref):
    p = p_ref[...]
    lane = lax.broadcasted_iota(jnp.int32, p.shape, 2)
    p0 = p[:, :, 0:1]
    po_ref[...] = jnp.where(
        lane == 0, p, p * (1.0 - _ERRP) + (1.0 - p - p0) * (_ERRP / (_V - 2))
    )
    pc_ref[...] = p
    mc_ref[...] = m_ref[...]


def _tc_rest(probs, messages):
    B, L, V = probs.shape
    nb = 64
    spec = pl.BlockSpec((nb, L, V), lambda i: (i, 0, 0))
    out_sds = jax.ShapeDtypeStruct((B, L, V), jnp.float32)
    return pl.pallas_call(
        _tc_rest_body,
        grid=(B // nb,),
        in_specs=[spec, spec],
        out_specs=[spec, spec, spec],
        out_shape=[out_sds, out_sds, out_sds],
    )(probs, messages)


def kernel(messages, probs):
    B, L, V = messages.shape
    bl = B * L
    with jax.ensure_compile_time_eval():
        packed = _packed_dest_table(bl).reshape(bl * _LANES)

    m_flat = messages.reshape(bl * V)
    mo = _sc_scatter(m_flat, packed)
    po, pc, mc = _tc_rest(probs, messages)

    eos = jnp.zeros((B, L), jnp.float32)
    return (mo.reshape(B, L, V), mc, po, pc, eos)


# speedup vs baseline: 1.3210x; 1.3210x over previous
import functools

import jax
import jax.numpy as jnp
from jax import lax
from jax.experimental import pallas as pl
from jax.experimental.pallas import tpu as pltpu
from jax.experimental.pallas import tpu_sc as plsc

_ERRP = 0.05
_V = 64
_NC = 2
_NS = 16
_LANES = 16
_CHUNK_ROWS = 200


def _packed_dest_table(bl):
    key = jax.random.key(42)
    k1, k2 = jax.random.split(key)
    tm = jax.random.uniform(k1, (bl, _V - 1)) < _ERRP
    repl_ids = jax.random.randint(k2, (bl, _V - 1), 0, _V - 2)
    s = jnp.arange(1, _V, dtype=jnp.int32)[None, :]
    repl_sym = jnp.where(repl_ids + 1 < s, repl_ids + 1, repl_ids + 2)
    dest = jnp.where(tm, repl_sym, s).astype(jnp.int32)
    dest = jnp.concatenate([jnp.zeros((bl, 1), jnp.int32), dest], axis=1)
    d4 = dest.reshape(bl, 4, _LANES)
    shifts = jnp.array([0, 8, 16, 24], jnp.int32)[None, :, None]
    return jnp.sum(d4 << shifts, axis=1, dtype=jnp.int32)


def _sc_scatter_body(
    m_hbm, pk_hbm, out_hbm,
    val0, val1, pk0, pk1, acc0, acc1,
    vs0, vs1, ps0, ps1, os0, os1,
):
    nw = _NC * _NS
    wid = lax.axis_index("s") * _NC + lax.axis_index("c")
    total = m_hbm.shape[0]
    per_w = total // nw
    chunk = _CHUNK_ROWS * _V
    pk_chunk = _CHUNK_ROWS * _LANES
    n_chunks = per_w // chunk
    wrow0 = wid * (per_w // _V)

    val = (val0, val1)
    pkb = (pk0, pk1)
    acc = (acc0, acc1)
    vsem = (vs0, vs1)
    psem = (ps0, ps1)
    osem = (os0, os1)

    def in_val(c, b):
        row0 = wrow0 + c * _CHUNK_ROWS
        return pltpu.make_async_copy(
            m_hbm.at[pl.ds(row0 * _V, chunk)], val[b], vsem[b]
        )

    def in_pk(c, b):
        row0 = wrow0 + c * _CHUNK_ROWS
        return pltpu.make_async_copy(
            pk_hbm.at[pl.ds(row0 * _LANES, pk_chunk)], pkb[b], psem[b]
        )

    def out_acc(c, b):
        row0 = wrow0 + c * _CHUNK_ROWS
        return pltpu.make_async_copy(
            acc[b], out_hbm.at[pl.ds(row0 * _V, chunk)], osem[b]
        )

    for b in range(2):
        in_val(b, b).start()
        in_pk(b, b).start()

    def compute(b):
        def row_body(r, carry):
            rb = r * _V
            for q in range(_V // _LANES):
                acc[b][pl.ds(rb + q * _LANES, _LANES)] = jnp.zeros(
                    (_LANES,), jnp.float32
                )
            pk = pkb[b][pl.ds(r * _LANES, _LANES)]
            for q in range(_V // _LANES):
                idx = ((pk >> (8 * q)) & 63) + rb
                v = val[b][pl.ds(rb + q * _LANES, _LANES)]
                plsc.addupdate_scatter(acc[b], [idx], v)
            return carry

        lax.fori_loop(0, _CHUNK_ROWS, row_body, 0, unroll=4)

    def pair_body(g, carry):
        for b in range(2):
            c = 2 * g + b
            in_val(c, b).wait()
            in_pk(c, b).wait()

            @pl.when(c >= 2)
            def _drain():
                out_acc(c - 2, b).wait()

            compute(b)

            @pl.when(c + 2 < n_chunks)
            def _prefetch():
                in_val(c + 2, b).start()
                in_pk(c + 2, b).start()

            out_acc(c, b).start()
        return carry

    lax.fori_loop(0, n_chunks // 2, pair_body, 0)
    for b in range(2):
        out_acc(n_chunks - 2 + b, b).wait()


def _sc_scatter(m_flat, packed_dest):
    total = m_flat.shape[0]
    chunk = _CHUNK_ROWS * _V
    mesh = plsc.VectorSubcoreMesh(core_axis_name="c", subcore_axis_name="s")
    return pl.kernel(
        _sc_scatter_body,
        mesh=mesh,
        out_type=jax.ShapeDtypeStruct((total,), jnp.float32),
        scratch_types=[
            pltpu.VMEM((chunk,), jnp.float32),
            pltpu.VMEM((chunk,), jnp.float32),
            pltpu.VMEM((_CHUNK_ROWS * _LANES,), jnp.int32),
            pltpu.VMEM((_CHUNK_ROWS * _LANES,), jnp.int32),
            pltpu.VMEM((chunk,), jnp.float32),
            pltpu.VMEM((chunk,), jnp.float32),
            pltpu.SemaphoreType.DMA,
            pltpu.SemaphoreType.DMA,
            pltpu.SemaphoreType.DMA,
            pltpu.SemaphoreType.DMA,
            pltpu.SemaphoreType.DMA,
            pltpu.SemaphoreType.DMA,
        ],
        compiler_params=pltpu.CompilerParams(needs_layout_passes=False),
    )(m_flat, packed_dest)


def _tc_probs_body(p_ref, po_ref):
    p = p_ref[...]
    lane = lax.broadcasted_iota(jnp.int32, p.shape, 2)
    p0 = p[:, :, 0:1]
    po_ref[...] = jnp.where(
        lane == 0, p, p * (1.0 - _ERRP) + (1.0 - p - p0) * (_ERRP / (_V - 2))
    )


def _tc_probs(probs):
    B, L, V = probs.shape
    nb = 64
    spec = pl.BlockSpec((nb, L, V), lambda i: (i, 0, 0))
    return pl.pallas_call(
        _tc_probs_body,
        grid=(B // nb,),
        in_specs=[spec],
        out_specs=spec,
        out_shape=jax.ShapeDtypeStruct((B, L, V), jnp.float32),
    )(probs)


def kernel(messages, probs):
    B, L, V = messages.shape
    bl = B * L
    with jax.ensure_compile_time_eval():
        packed = _packed_dest_table(bl).reshape(bl * _LANES)

    m_flat = messages.reshape(bl * V)
    mo = _sc_scatter(m_flat, packed)
    po = _tc_probs(probs)

    eos = jnp.zeros((B, L), jnp.float32)
    return (mo.reshape(B, L, V), messages, po, probs, eos)
